# baseline (speedup 1.0000x reference)
import functools

import jax
import jax.numpy as jnp
from jax import lax
from jax.experimental import pallas as pl
from jax.experimental.pallas import tpu as pltpu
from jax.experimental.pallas import tpu_sc as plsc

N = 10000
D = 256
HALF = 128
E = 160000
NC = 2
NS = 16
CHUNK = 128
MSG_CHUNKS = 80
E_PAD = MSG_CHUNKS * NS * CHUNK
DEG_CHUNKS = E_PAD // (NC * NS * CHUNK)
RPT = 632
RPT_LAST = N - 15 * RPT
ACC_ROWS = N + NS
HCH = MSG_CHUNKS // 2

_f32 = jnp.float32
_mesh = functools.partial(
    plsc.VectorSubcoreMesh, core_axis_name="c", subcore_axis_name="s")


def _deg_body(dst_hbm, zeros_hbm, ones_hbm, deg_out, sh_deg, ones_v, idx_v,
              dsem):
    cid = lax.axis_index("c")
    sid = lax.axis_index("s")
    row0 = sid * RPT

    def _split_copy(fn15, fn):
        pl.when(sid < 15)(fn)
        pl.when(sid == 15)(fn15)

    _split_copy(
        lambda: pltpu.sync_copy(zeros_hbm.at[pl.ds(15 * RPT, RPT_LAST + NS)],
                                sh_deg.at[pl.ds(15 * RPT, RPT_LAST + NS)]),
        lambda: pltpu.sync_copy(zeros_hbm.at[pl.ds(row0, RPT)],
                                sh_deg.at[pl.ds(row0, RPT)]))
    pltpu.sync_copy(ones_hbm, ones_v)
    pltpu.sync_copy(dst_hbm.at[cid * NS + sid], idx_v)
    plsc.subcore_barrier()

    def fire(k, carry):
        pltpu.async_copy(ones_v, sh_deg.at[idx_v.at[k]], dsem, add=True)
        return carry

    def drain(k, carry):
        pltpu.make_async_copy(ones_v, sh_deg.at[idx_v.at[k]], dsem).wait()
        return carry

    lax.fori_loop(0, DEG_CHUNKS, fire, 0)
    lax.fori_loop(0, DEG_CHUNKS, drain, 0)
    plsc.subcore_barrier()
    _split_copy(
        lambda: pltpu.sync_copy(sh_deg.at[pl.ds(15 * RPT, RPT_LAST)],
                                deg_out.at[cid, pl.ds(15 * RPT, RPT_LAST)]),
        lambda: pltpu.sync_copy(sh_deg.at[pl.ds(row0, RPT)],
                                deg_out.at[cid, pl.ds(row0, RPT)]))


def _deg_kernel(dst_pad, zeros128, ones128):
    return pl.kernel(
        _deg_body,
        out_type=jax.ShapeDtypeStruct((NC, ACC_ROWS, HALF), _f32),
        mesh=_mesh(),
        scratch_types=[
            pltpu.VMEM_SHARED((ACC_ROWS, HALF), _f32),
            pltpu.VMEM((CHUNK, HALF), _f32),
            pltpu.VMEM((DEG_CHUNKS, CHUNK), jnp.int32),
            pltpu.SemaphoreType.DMA,
        ],
    )(dst_pad, zeros128, ones128)


def _msg_body(za, zb, src_hbm, dst_hbm, ua, ub,
              sh_u, rows_v, sidx_v, didx_v, gsem0, gsem1, ssem0, ssem1):
    cid = lax.axis_index("c")
    sid = lax.axis_index("s")

    def run(z_hbm, u_hbm):
        row0 = sid * RPT
        pl.when(sid < 15)(lambda: pltpu.sync_copy(
            z_hbm.at[pl.ds(row0, RPT)], sh_u.at[pl.ds(row0, RPT)]))
        pl.when(sid == 15)(lambda: pltpu.sync_copy(
            z_hbm.at[pl.ds(15 * RPT, RPT_LAST)],
            sh_u.at[pl.ds(15 * RPT, RPT_LAST)]))
        plsc.subcore_barrier()

        def gather_start(k, buf, sem):
            pltpu.async_copy(z_hbm.at[sidx_v.at[k]], rows_v.at[buf], sem)

        def gather_wait(k, buf, sem):
            pltpu.make_async_copy(
                z_hbm.at[sidx_v.at[k]], rows_v.at[buf], sem).wait()

        def scatter_start(k, buf, sem):
            pltpu.async_copy(rows_v.at[buf], sh_u.at[didx_v.at[k]], sem,
                             add=True)

        def scatter_wait(k, buf, sem):
            pltpu.make_async_copy(rows_v.at[buf], sh_u.at[didx_v.at[k]],
                                  sem).wait()

        for p in range(2):
            pltpu.sync_copy(src_hbm.at[sid, pl.ds(p * HCH, HCH)], sidx_v)
            pltpu.sync_copy(dst_hbm.at[sid, pl.ds(p * HCH, HCH)], didx_v)
            gather_start(0, 0, gsem0)
            gather_start(1, 1, gsem1)

            def step(j, carry):
                k0 = 2 * j
                gather_wait(k0, 0, gsem0)
                scatter_start(k0, 0, ssem0)
                gather_wait(k0 + 1, 1, gsem1)
                scatter_start(k0 + 1, 1, ssem1)

                def refill():
                    scatter_wait(k0, 0, ssem0)
                    gather_start(k0 + 2, 0, gsem0)
                    scatter_wait(k0 + 1, 1, ssem1)
                    gather_start(k0 + 3, 1, gsem1)

                pl.when(j < HCH // 2 - 1)(refill)
                return carry

            lax.fori_loop(0, HCH // 2, step, 0)
            scatter_wait(HCH - 2, 0, ssem0)
            scatter_wait(HCH - 1, 1, ssem1)
        plsc.subcore_barrier()
        pl.when(sid < 15)(lambda: pltpu.sync_copy(
            sh_u.at[pl.ds(row0, RPT)], u_hbm.at[pl.ds(row0, RPT)]))
        pl.when(sid == 15)(lambda: pltpu.sync_copy(
            sh_u.at[pl.ds(15 * RPT, RPT_LAST)],
            u_hbm.at[pl.ds(15 * RPT, RPT_LAST)]))

    pl.when(cid == 0)(lambda: run(za, ua))
    pl.when(cid == 1)(lambda: run(zb, ub))


def _msg_kernel(za, zb, src3, dst3):
    return pl.kernel(
        _msg_body,
        out_type=(jax.ShapeDtypeStruct((N, HALF), _f32),
                  jax.ShapeDtypeStruct((N, HALF), _f32)),
        mesh=_mesh(),
        scratch_types=[
            pltpu.VMEM_SHARED((ACC_ROWS, HALF), _f32),
            pltpu.VMEM((2, CHUNK, HALF), _f32),
            pltpu.VMEM((HCH, CHUNK), jnp.int32),
            pltpu.VMEM((HCH, CHUNK), jnp.int32),
            pltpu.SemaphoreType.DMA,
            pltpu.SemaphoreType.DMA,
            pltpu.SemaphoreType.DMA,
            pltpu.SemaphoreType.DMA,
        ],
    )(za, zb, src3, dst3)


_R = 2000


def _dinv(dega_ref, degb_ref):
    deg = dega_ref[0, :, 0:1] + degb_ref[0, :, 0:1] + 1.0
    return lax.rsqrt(deg)


def _lin1_body(x_ref, w_ref, dega_ref, degb_ref, za_ref, zb_ref):
    z = jnp.dot(x_ref[...], w_ref[...], preferred_element_type=_f32)
    z = z * _dinv(dega_ref, degb_ref)
    za_ref[...] = z[:, :HALF]
    zb_ref[...] = z[:, HALF:]


def _lin2_body(ua_ref, ub_ref, dega_ref, degb_ref, b_ref, w_ref,
               za_ref, zb_ref):
    dinv = _dinv(dega_ref, degb_ref)
    u = jnp.concatenate([ua_ref[...], ub_ref[...]], axis=1)
    h = jnp.maximum(u * dinv + b_ref[...], 0.0)
    z = jnp.dot(h, w_ref[...], preferred_element_type=_f32) * dinv
    za_ref[...] = z[:, :HALF]
    zb_ref[...] = z[:, HALF:]


def _out_body(ua_ref, ub_ref, dega_ref, degb_ref, b_ref, o_ref):
    dinv = _dinv(dega_ref, degb_ref)
    u = jnp.concatenate([ua_ref[...], ub_ref[...]], axis=1)
    o_ref[...] = u * dinv + b_ref[...]


def _deg_specs():
    return [
        pl.BlockSpec((1, _R, HALF), lambda i: (0, i, 0)),
        pl.BlockSpec((1, _R, HALF), lambda i: (1, i, 0)),
    ]


def _lin1(x, w1, deg):
    return pl.pallas_call(
        _lin1_body,
        grid=(N // _R,),
        in_specs=[
            pl.BlockSpec((_R, D), lambda i: (i, 0)),
            pl.BlockSpec((D, D), lambda i: (0, 0)),
            *_deg_specs(),
        ],
        out_specs=(pl.BlockSpec((_R, HALF), lambda i: (i, 0)),
                   pl.BlockSpec((_R, HALF), lambda i: (i, 0))),
        out_shape=(jax.ShapeDtypeStruct((N, HALF), _f32),
                   jax.ShapeDtypeStruct((N, HALF), _f32)),
    )(x, w1, deg, deg)


def _lin2(ua, ub, deg, b1, w2):
    return pl.pallas_call(
        _lin2_body,
        grid=(N // _R,),
        in_specs=[
            pl.BlockSpec((_R, HALF), lambda i: (i, 0)),
            pl.BlockSpec((_R, HALF), lambda i: (i, 0)),
            *_deg_specs(),
            pl.BlockSpec((1, D), lambda i: (0, 0)),
            pl.BlockSpec((D, D), lambda i: (0, 0)),
        ],
        out_specs=(pl.BlockSpec((_R, HALF), lambda i: (i, 0)),
                   pl.BlockSpec((_R, HALF), lambda i: (i, 0))),
        out_shape=(jax.ShapeDtypeStruct((N, HALF), _f32),
                   jax.ShapeDtypeStruct((N, HALF), _f32)),
    )(ua, ub, deg, deg, b1.reshape(1, D), w2)


def _finish(ua, ub, deg, b2):
    return pl.pallas_call(
        _out_body,
        grid=(N // _R,),
        in_specs=[
            pl.BlockSpec((_R, HALF), lambda i: (i, 0)),
            pl.BlockSpec((_R, HALF), lambda i: (i, 0)),
            *_deg_specs(),
            pl.BlockSpec((1, D), lambda i: (0, 0)),
        ],
        out_specs=pl.BlockSpec((_R, D), lambda i: (i, 0)),
        out_shape=jax.ShapeDtypeStruct((N, D), _f32),
    )(ua, ub, deg, deg, b2.reshape(1, D))


def kernel(features, graph, W1, b1, W2, b2):
    src = graph[0].astype(jnp.int32)
    dst = graph[1].astype(jnp.int32)
    pad = E_PAD - E
    ar = jnp.arange(pad, dtype=jnp.int32)
    src_pad = jnp.concatenate([src, (ar * 37) % N])
    dst_pad = jnp.concatenate([dst, N + (ar % NS)])

    zeros128 = jnp.zeros((ACC_ROWS, HALF), _f32)
    ones128 = jnp.ones((CHUNK, HALF), _f32)

    dst_deg = dst_pad.reshape(NC * NS, DEG_CHUNKS, CHUNK)
    src3 = src_pad.reshape(NS, MSG_CHUNKS, CHUNK)
    dst3 = dst_pad.reshape(NS, MSG_CHUNKS, CHUNK)

    deg = _deg_kernel(dst_deg, zeros128, ones128)
    za, zb = _lin1(features, W1, deg)
    ua, ub = _msg_kernel(za, zb, src3, dst3)
    za2, zb2 = _lin2(ua, ub, deg, b1, W2)
    ua2, ub2 = _msg_kernel(za2, zb2, src3, dst3)
    return _finish(ua2, ub2, deg, b2)


# speedup vs baseline: 1.2198x; 1.2198x over previous
import functools

import jax
import jax.numpy as jnp
from jax import lax
from jax.experimental import pallas as pl
from jax.experimental.pallas import tpu as pltpu
from jax.experimental.pallas import tpu_sc as plsc

N = 10000
D = 256
HALF = 128
E = 160000
NC = 2
NS = 16
CHUNK = 128
MSG_CHUNKS = 80
E_PAD = MSG_CHUNKS * NS * CHUNK
DEG_CHUNKS = E_PAD // (NC * NS * CHUNK)
RPT = 632
RPT_LAST = N - 15 * RPT
ACC_ROWS = N + NS
HCH = MSG_CHUNKS // 2

_f32 = jnp.float32
_mesh = functools.partial(
    plsc.VectorSubcoreMesh, core_axis_name="c", subcore_axis_name="s")


def _deg_body(dst_hbm, zeros_hbm, ones_hbm, deg_out, sh_deg, ones_v, idx_v,
              dsem):
    cid = lax.axis_index("c")
    sid = lax.axis_index("s")
    row0 = sid * RPT

    def _split_copy(fn15, fn):
        pl.when(sid < 15)(fn)
        pl.when(sid == 15)(fn15)

    _split_copy(
        lambda: pltpu.sync_copy(zeros_hbm.at[pl.ds(15 * RPT, RPT_LAST + NS)],
                                sh_deg.at[pl.ds(15 * RPT, RPT_LAST + NS)]),
        lambda: pltpu.sync_copy(zeros_hbm.at[pl.ds(row0, RPT)],
                                sh_deg.at[pl.ds(row0, RPT)]))
    pltpu.sync_copy(ones_hbm, ones_v)
    pltpu.sync_copy(dst_hbm.at[cid * NS + sid], idx_v)
    plsc.subcore_barrier()

    def fire(k, carry):
        pltpu.async_copy(ones_v, sh_deg.at[idx_v.at[k]], dsem, add=True)
        return carry

    def drain(k, carry):
        pltpu.make_async_copy(ones_v, sh_deg.at[idx_v.at[k]], dsem).wait()
        return carry

    lax.fori_loop(0, DEG_CHUNKS, fire, 0)
    lax.fori_loop(0, DEG_CHUNKS, drain, 0)
    plsc.subcore_barrier()
    _split_copy(
        lambda: pltpu.sync_copy(sh_deg.at[pl.ds(15 * RPT, RPT_LAST)],
                                deg_out.at[cid, pl.ds(15 * RPT, RPT_LAST)]),
        lambda: pltpu.sync_copy(sh_deg.at[pl.ds(row0, RPT)],
                                deg_out.at[cid, pl.ds(row0, RPT)]))


def _deg_kernel(dst_pad, zeros128, ones128):
    return pl.kernel(
        _deg_body,
        out_type=jax.ShapeDtypeStruct((NC, ACC_ROWS, HALF), _f32),
        mesh=_mesh(),
        scratch_types=[
            pltpu.VMEM_SHARED((ACC_ROWS, HALF), _f32),
            pltpu.VMEM((CHUNK, HALF), _f32),
            pltpu.VMEM((DEG_CHUNKS, CHUNK), jnp.int32),
            pltpu.SemaphoreType.DMA,
        ],
    )(dst_pad, zeros128, ones128)


def _msg_body(za, zb, src_hbm, dst_hbm, ua, ub,
              sh_u, rows_v, sidx_v, didx_v, gsem0, gsem1):
    cid = lax.axis_index("c")
    sid = lax.axis_index("s")

    def run(z_hbm, u_hbm):
        row0 = sid * RPT
        pl.when(sid < 15)(lambda: pltpu.sync_copy(
            z_hbm.at[pl.ds(row0, RPT)], sh_u.at[pl.ds(row0, RPT)]))
        pl.when(sid == 15)(lambda: pltpu.sync_copy(
            z_hbm.at[pl.ds(15 * RPT, RPT_LAST)],
            sh_u.at[pl.ds(15 * RPT, RPT_LAST)]))
        plsc.subcore_barrier()

        def gather_start(k, buf, sem):
            pltpu.async_copy(z_hbm.at[sidx_v.at[k]], rows_v.at[buf], sem)

        def gather_wait(k, buf, sem):
            pltpu.make_async_copy(
                z_hbm.at[sidx_v.at[k]], rows_v.at[buf], sem).wait()

        for p in range(2):
            pltpu.sync_copy(src_hbm.at[sid, pl.ds(p * HCH, HCH)], sidx_v)
            pltpu.sync_copy(dst_hbm.at[sid, pl.ds(p * HCH, HCH)], didx_v)
            gather_start(0, 0, gsem0)
            gather_start(1, 1, gsem1)

            def step(j, carry):
                k0 = 2 * j
                gather_wait(k0, 0, gsem0)
                pltpu.sync_copy(rows_v.at[0], sh_u.at[didx_v.at[k0]],
                                add=True)
                pl.when(j < HCH // 2 - 1)(
                    lambda: gather_start(k0 + 2, 0, gsem0))
                gather_wait(k0 + 1, 1, gsem1)
                pltpu.sync_copy(rows_v.at[1], sh_u.at[didx_v.at[k0 + 1]],
                                add=True)
                pl.when(j < HCH // 2 - 1)(
                    lambda: gather_start(k0 + 3, 1, gsem1))
                return carry

            lax.fori_loop(0, HCH // 2, step, 0)
        plsc.subcore_barrier()
        pl.when(sid < 15)(lambda: pltpu.sync_copy(
            sh_u.at[pl.ds(row0, RPT)], u_hbm.at[pl.ds(row0, RPT)]))
        pl.when(sid == 15)(lambda: pltpu.sync_copy(
            sh_u.at[pl.ds(15 * RPT, RPT_LAST)],
            u_hbm.at[pl.ds(15 * RPT, RPT_LAST)]))

    pl.when(cid == 0)(lambda: run(za, ua))
    pl.when(cid == 1)(lambda: run(zb, ub))


def _msg_kernel(za, zb, src3, dst3):
    return pl.kernel(
        _msg_body,
        out_type=(jax.ShapeDtypeStruct((N, HALF), _f32),
                  jax.ShapeDtypeStruct((N, HALF), _f32)),
        mesh=_mesh(),
        scratch_types=[
            pltpu.VMEM_SHARED((ACC_ROWS, HALF), _f32),
            pltpu.VMEM((2, CHUNK, HALF), _f32),
            pltpu.VMEM((HCH, CHUNK), jnp.int32),
            pltpu.VMEM((HCH, CHUNK), jnp.int32),
            pltpu.SemaphoreType.DMA,
            pltpu.SemaphoreType.DMA,
        ],
    )(za, zb, src3, dst3)


_R = 2000


def _dinv(dega_ref, degb_ref):
    deg = dega_ref[0, :, 0:1] + degb_ref[0, :, 0:1] + 1.0
    return lax.rsqrt(deg)


def _lin1_body(x_ref, w_ref, dega_ref, degb_ref, za_ref, zb_ref):
    z = jnp.dot(x_ref[...], w_ref[...], preferred_element_type=_f32)
    z = z * _dinv(dega_ref, degb_ref)
    za_ref[...] = z[:, :HALF]
    zb_ref[...] = z[:, HALF:]


def _lin2_body(ua_ref, ub_ref, dega_ref, degb_ref, b_ref, w_ref,
               za_ref, zb_ref):
    dinv = _dinv(dega_ref, degb_ref)
    u = jnp.concatenate([ua_ref[...], ub_ref[...]], axis=1)
    h = jnp.maximum(u * dinv + b_ref[...], 0.0)
    z = jnp.dot(h, w_ref[...], preferred_element_type=_f32) * dinv
    za_ref[...] = z[:, :HALF]
    zb_ref[...] = z[:, HALF:]


def _out_body(ua_ref, ub_ref, dega_ref, degb_ref, b_ref, o_ref):
    dinv = _dinv(dega_ref, degb_ref)
    u = jnp.concatenate([ua_ref[...], ub_ref[...]], axis=1)
    o_ref[...] = u * dinv + b_ref[...]


def _deg_specs():
    return [
        pl.BlockSpec((1, _R, HALF), lambda i: (0, i, 0)),
        pl.BlockSpec((1, _R, HALF), lambda i: (1, i, 0)),
    ]


def _lin1(x, w1, deg):
    return pl.pallas_call(
        _lin1_body,
        grid=(N // _R,),
        in_specs=[
            pl.BlockSpec((_R, D), lambda i: (i, 0)),
            pl.BlockSpec((D, D), lambda i: (0, 0)),
            *_deg_specs(),
        ],
        out_specs=(pl.BlockSpec((_R, HALF), lambda i: (i, 0)),
                   pl.BlockSpec((_R, HALF), lambda i: (i, 0))),
        out_shape=(jax.ShapeDtypeStruct((N, HALF), _f32),
                   jax.ShapeDtypeStruct((N, HALF), _f32)),
    )(x, w1, deg, deg)


def _lin2(ua, ub, deg, b1, w2):
    return pl.pallas_call(
        _lin2_body,
        grid=(N // _R,),
        in_specs=[
            pl.BlockSpec((_R, HALF), lambda i: (i, 0)),
            pl.BlockSpec((_R, HALF), lambda i: (i, 0)),
            *_deg_specs(),
            pl.BlockSpec((1, D), lambda i: (0, 0)),
            pl.BlockSpec((D, D), lambda i: (0, 0)),
        ],
        out_specs=(pl.BlockSpec((_R, HALF), lambda i: (i, 0)),
                   pl.BlockSpec((_R, HALF), lambda i: (i, 0))),
        out_shape=(jax.ShapeDtypeStruct((N, HALF), _f32),
                   jax.ShapeDtypeStruct((N, HALF), _f32)),
    )(ua, ub, deg, deg, b1.reshape(1, D), w2)


def _finish(ua, ub, deg, b2):
    return pl.pallas_call(
        _out_body,
        grid=(N // _R,),
        in_specs=[
            pl.BlockSpec((_R, HALF), lambda i: (i, 0)),
            pl.BlockSpec((_R, HALF), lambda i: (i, 0)),
            *_deg_specs(),
            pl.BlockSpec((1, D), lambda i: (0, 0)),
        ],
        out_specs=pl.BlockSpec((_R, D), lambda i: (i, 0)),
        out_shape=jax.ShapeDtypeStruct((N, D), _f32),
    )(ua, ub, deg, deg, b2.reshape(1, D))


def kernel(features, graph, W1, b1, W2, b2):
    src = graph[0].astype(jnp.int32)
    dst = graph[1].astype(jnp.int32)
    pad = E_PAD - E
    ar = jnp.arange(pad, dtype=jnp.int32)
    src_pad = jnp.concatenate([src, (ar * 37) % N])
    dst_pad = jnp.concatenate([dst, N + (ar % NS)])

    zeros128 = jnp.zeros((ACC_ROWS, HALF), _f32)
    ones128 = jnp.ones((CHUNK, HALF), _f32)

    dst_deg = dst_pad.reshape(NC * NS, DEG_CHUNKS, CHUNK)
    src3 = src_pad.reshape(NS, MSG_CHUNKS, CHUNK)
    dst3 = dst_pad.reshape(NS, MSG_CHUNKS, CHUNK)

    deg = _deg_kernel(dst_deg, zeros128, ones128)
    za, zb = _lin1(features, W1, deg)
    ua, ub = _msg_kernel(za, zb, src3, dst3)
    za2, zb2 = _lin2(ua, ub, deg, b1, W2)
    ua2, ub2 = _msg_kernel(za2, zb2, src3, dst3)
    return _finish(ua2, ub2, deg, b2)
